# baseline (speedup 1.0000x reference)
import functools

import jax
import jax.numpy as jnp
from jax import lax
from jax.experimental import pallas as pl
from jax.experimental.pallas import tpu as pltpu
from jax.experimental.pallas import tpu_sc as plsc

_NC = 2
_NS = 16
_NW = _NC * _NS

_CH = 128
_ROW_BLK = 1280

def _cdiv(a, b):
  return (a + b - 1) // b


def _make_sc_agg(n_pad, n_acc, n_chunks, d):
  rows_per_tile = n_acc // _NS
  zero_chunks = rows_per_tile // 8
  out_full = rows_per_tile // _CH
  out_tail = rows_per_tile - out_full * _CH
  nk = d // 16
  mesh = plsc.VectorSubcoreMesh(core_axis_name="c", subcore_axis_name="s",
                                num_cores=_NC, num_subcores=_NS)

  @functools.partial(
      pl.kernel,
      out_type=jax.ShapeDtypeStruct((_NC, n_pad, d), jnp.float32),
      mesh=mesh,
      scratch_types=[
          pltpu.VMEM((n_chunks * 3, _CH), jnp.int32),
          pltpu.VMEM((_CH, d), jnp.float32),
          pltpu.VMEM((8, d), jnp.float32),
          pltpu.VMEM_SHARED((n_acc, d), jnp.float32),
          pltpu.SemaphoreType.DMA,
      ],
  )
  def sc_agg(h_hbm, meta_hbm, out_hbm, meta_v, rows_v, zbuf, agg_sh, sem):
    cid = lax.axis_index("c")
    sid = lax.axis_index("s")
    wid = cid * _NS + sid

    pltpu.sync_copy(meta_hbm.at[wid], meta_v)

    zk = jnp.zeros((16,), jnp.float32)
    for r in range(8):
      for c in range(nk):
        zbuf[r, pl.ds(c * 16, 16)] = zk

    def zloop(r, carry):
      pltpu.async_copy(
          zbuf, agg_sh.at[pl.ds((sid * zero_chunks + r) * 8, 8)], sem)
      return carry

    lax.fori_loop(0, zero_chunks, zloop, 0)

    def zdrain(r, carry):
      pltpu.make_async_copy(
          zbuf, agg_sh.at[pl.ds((sid * zero_chunks + r) * 8, 8)], sem).wait()
      return carry

    lax.fori_loop(0, zero_chunks, zdrain, 0)
    plsc.subcore_barrier()

    def eloop(j, accs):
      j3 = j * 3
      pltpu.sync_copy(h_hbm.at[meta_v.at[j3]], rows_v)

      def gloop(g, accs):
        sf = jnp.astype(meta_v[j3 + 2, pl.ds(g * 16, 16)], jnp.float32)
        for ei in range(16):
          s = sf[ei]
          row = g * 16 + ei
          accs = tuple(
              accs[k] * s + rows_v[row, pl.ds(k * 16, 16)]
              for k in range(nk))
          for k in range(nk):
            rows_v[row, pl.ds(k * 16, 16)] = accs[k]
        return accs

      accs = lax.fori_loop(0, _CH // 16, gloop, accs)
      pltpu.sync_copy(rows_v, agg_sh.at[meta_v.at[j3 + 1]], add=True)
      return accs

    zero_accs = tuple(jnp.zeros((16,), jnp.float32) for _ in range(nk))
    lax.fori_loop(0, n_chunks, eloop, zero_accs)
    plsc.subcore_barrier()

    def oloop(k, carry):
      base = sid * rows_per_tile + k * _CH
      pltpu.async_copy(agg_sh.at[pl.ds(base, _CH)],
                       out_hbm.at[cid].at[pl.ds(base, _CH)], sem)
      return carry

    lax.fori_loop(0, out_full, oloop, 0)
    tb = sid * rows_per_tile + out_full * _CH
    if out_tail:
      pltpu.async_copy(agg_sh.at[pl.ds(tb, out_tail)],
                       out_hbm.at[cid].at[pl.ds(tb, out_tail)], sem)

    def odrain(k, carry):
      base = sid * rows_per_tile + k * _CH
      pltpu.make_async_copy(agg_sh.at[pl.ds(base, _CH)],
                            out_hbm.at[cid].at[pl.ds(base, _CH)], sem).wait()
      return carry

    lax.fori_loop(0, out_full, odrain, 0)
    if out_tail:
      pltpu.make_async_copy(agg_sh.at[pl.ds(tb, out_tail)],
                            out_hbm.at[cid].at[pl.ds(tb, out_tail)],
                            sem).wait()

  return sc_agg


def _dot_t(x, w):
  return lax.dot_general(x, w, (((1,), (1,)), ((), ())),
                         preferred_element_type=jnp.float32)


def _dense0_body(x_ref, w_ref, b_ref, o_ref):
  o_ref[...] = jnp.tanh(_dot_t(x_ref[...], w_ref[...]) + b_ref[...])


def _layer_body(agg_ref, h_ref, wrel_ref, wroot_ref, b_ref, o_ref):
  a = agg_ref[0] + agg_ref[1]
  o_ref[...] = jnp.tanh((_dot_t(a, wrel_ref[...]) + b_ref[...]) +
                        _dot_t(h_ref[...], wroot_ref[...]))


def _final_body(h_ref, w_ref, b_ref, o_ref):
  o_ref[...] = jnp.maximum(
      _dot_t(h_ref[...], w_ref[...]) + b_ref[...], 0.0)


def _row_blocked(body, n_pad, d, in_specs):
  grid = (n_pad // _ROW_BLK,)
  return pl.pallas_call(
      body,
      grid=grid,
      in_specs=in_specs,
      out_specs=pl.BlockSpec((_ROW_BLK, d), lambda i: (i, 0)),
      out_shape=jax.ShapeDtypeStruct((n_pad, d), jnp.float32),
  )


def _mat_spec(d):
  return pl.BlockSpec((d, d), lambda i: (0, 0))


def _bias_spec(d):
  return pl.BlockSpec((1, d), lambda i: (0, 0))


def kernel(x, edge_index, W1, b1, Wrel, brel, Wroot, W2, b2):
  n, d = x.shape
  e = edge_index.shape[1]
  l = Wrel.shape[0]

  n_pad = _cdiv(n, _NS * _CH) * _NS * _CH
  n_chunks = _cdiv(e, _NW * _CH)
  t_edges = n_chunks * _CH
  n_acc = _cdiv(n + 2 + 3 * _NW, _NS * 8) * _NS * 8
  if n_acc > n_pad:
    n_pad = _cdiv(n_acc, _NS * _CH) * _NS * _CH

  x_pad = jnp.pad(x, ((0, n_pad - n), (0, 0)))

  order = jnp.argsort(edge_index[1], stable=True)
  ds = edge_index[1][order]
  ss = edge_index[0][order]

  base = e // _NW
  slack = t_edges - base
  q = jnp.arange(_NW, dtype=jnp.int32) * base
  rs = jnp.searchsorted(ds, ds[q], side="left").astype(jnp.int32)
  starts = jnp.where(q - rs <= slack - 1, rs, q)
  pos = jnp.arange(e, dtype=jnp.int32)
  ti = jnp.searchsorted(starts, pos, side="right").astype(jnp.int32) - 1
  slot = ti * t_edges + (pos - starts[ti])

  e_pad = _NW * t_edges
  ds_p = jnp.full((e_pad,), n, jnp.int32).at[slot].set(ds)
  ss_p = jnp.zeros((e_pad,), jnp.int32).at[slot].set(ss)
  d2 = ds_p.reshape(_NW, t_edges)
  s2 = ss_p.reshape(_NW, t_edges)
  col = jnp.arange(t_edges, dtype=jnp.int32)[None, :]
  prev = jnp.roll(d2, 1, axis=1)
  nxt = jnp.roll(d2, -1, axis=1)
  same2 = ((d2 == prev) & (col != 0)).astype(jnp.int32)
  is_end = (d2 != nxt) | (col == t_edges - 1)
  tile_id = jnp.arange(_NW, dtype=jnp.int32)[:, None]
  sink2 = n + 1 + tile_id * 3 + (col % 3)
  sdst2 = jnp.where(is_end, d2, sink2)

  meta3 = jnp.stack(
      [s2.reshape(_NW, n_chunks, _CH),
       sdst2.reshape(_NW, n_chunks, _CH),
       same2.reshape(_NW, n_chunks, _CH)],
      axis=2).reshape(_NW, n_chunks * 3, _CH)

  sc_agg = _make_sc_agg(n_pad, n_acc, n_chunks, d)

  row_spec = pl.BlockSpec((_ROW_BLK, d), lambda i: (i, 0))
  agg_spec = pl.BlockSpec((_NC, _ROW_BLK, d), lambda i: (0, i, 0))

  dense0 = _row_blocked(_dense0_body, n_pad, d,
                        [row_spec, _mat_spec(d), _bias_spec(d)])
  layer = _row_blocked(_layer_body, n_pad, d,
                       [agg_spec, row_spec, _mat_spec(d), _mat_spec(d),
                        _bias_spec(d)])
  final = _row_blocked(_final_body, n_pad, d,
                       [row_spec, _mat_spec(d), _bias_spec(d)])

  h = dense0(x_pad, W1, b1.reshape(1, d))
  for i in range(l):
    agg = sc_agg(h, meta3)
    h = layer(agg, h, Wrel[i], Wroot[i], brel[i].reshape(1, d))
  out = final(h, W2, b2.reshape(1, d))
  return out[:n]


# speedup vs baseline: 1.6200x; 1.6200x over previous
import functools

import jax
import jax.numpy as jnp
from jax import lax
from jax.experimental import pallas as pl
from jax.experimental.pallas import tpu as pltpu
from jax.experimental.pallas import tpu_sc as plsc

_NC = 2
_NS = 16
_NW = _NC * _NS

_CH = 128
_ROW_BLK = 1280

def _cdiv(a, b):
  return (a + b - 1) // b


def _make_sc_agg(n_pad, n_acc, n_chunks, d):
  rows_per_tile = n_acc // _NS
  zero_chunks = rows_per_tile // 8
  out_full = rows_per_tile // _CH
  out_tail = rows_per_tile - out_full * _CH
  nk = d // 16
  mesh = plsc.VectorSubcoreMesh(core_axis_name="c", subcore_axis_name="s",
                                num_cores=_NC, num_subcores=_NS)

  @functools.partial(
      pl.kernel,
      out_type=jax.ShapeDtypeStruct((_NC, n_pad, d), jnp.float32),
      mesh=mesh,
      scratch_types=[
          pltpu.VMEM((n_chunks * 3, _CH), jnp.int32),
          pltpu.VMEM((_CH, d), jnp.float32),
          pltpu.VMEM((8, d), jnp.float32),
          pltpu.VMEM_SHARED((n_acc, d), jnp.float32),
          pltpu.SemaphoreType.DMA,
      ],
  )
  def sc_agg(h_hbm, meta_hbm, out_hbm, meta_v, rows_v, zbuf, agg_sh, sem):
    cid = lax.axis_index("c")
    sid = lax.axis_index("s")
    wid = cid * _NS + sid

    pltpu.sync_copy(meta_hbm.at[wid], meta_v)

    zk = jnp.zeros((16,), jnp.float32)
    for r in range(8):
      for c in range(nk):
        zbuf[r, pl.ds(c * 16, 16)] = zk

    def zloop(r, carry):
      pltpu.async_copy(
          zbuf, agg_sh.at[pl.ds((sid * zero_chunks + r) * 8, 8)], sem)
      return carry

    lax.fori_loop(0, zero_chunks, zloop, 0)

    def zdrain(r, carry):
      pltpu.make_async_copy(
          zbuf, agg_sh.at[pl.ds((sid * zero_chunks + r) * 8, 8)], sem).wait()
      return carry

    lax.fori_loop(0, zero_chunks, zdrain, 0)
    plsc.subcore_barrier()

    def eloop(j, accs):
      j3 = j * 3
      pltpu.sync_copy(h_hbm.at[meta_v.at[j3]], rows_v)

      def gloop(g, accs):
        sf = jnp.astype(meta_v[j3 + 2, pl.ds(g * 16, 16)], jnp.float32)
        for ei in range(16):
          s = sf[ei]
          row = g * 16 + ei
          accs = tuple(
              accs[k] * s + rows_v[row, pl.ds(k * 16, 16)]
              for k in range(nk))
          for k in range(nk):
            rows_v[row, pl.ds(k * 16, 16)] = accs[k]
        return accs

      accs = lax.fori_loop(0, _CH // 16, gloop, accs)
      pltpu.sync_copy(rows_v, agg_sh.at[meta_v.at[j3 + 1]], add=True)
      return accs

    zero_accs = tuple(jnp.zeros((16,), jnp.float32) for _ in range(nk))
    lax.fori_loop(0, n_chunks, eloop, zero_accs)
    plsc.subcore_barrier()

    def oloop(k, carry):
      base = sid * rows_per_tile + k * _CH
      pltpu.async_copy(agg_sh.at[pl.ds(base, _CH)],
                       out_hbm.at[cid].at[pl.ds(base, _CH)], sem)
      return carry

    lax.fori_loop(0, out_full, oloop, 0)
    tb = sid * rows_per_tile + out_full * _CH
    if out_tail:
      pltpu.async_copy(agg_sh.at[pl.ds(tb, out_tail)],
                       out_hbm.at[cid].at[pl.ds(tb, out_tail)], sem)

    def odrain(k, carry):
      base = sid * rows_per_tile + k * _CH
      pltpu.make_async_copy(agg_sh.at[pl.ds(base, _CH)],
                            out_hbm.at[cid].at[pl.ds(base, _CH)], sem).wait()
      return carry

    lax.fori_loop(0, out_full, odrain, 0)
    if out_tail:
      pltpu.make_async_copy(agg_sh.at[pl.ds(tb, out_tail)],
                            out_hbm.at[cid].at[pl.ds(tb, out_tail)],
                            sem).wait()

  return sc_agg


def _dot_t(x, w):
  return lax.dot_general(x, w, (((1,), (1,)), ((), ())),
                         preferred_element_type=jnp.float32)


def _dense0_body(x_ref, w_ref, b_ref, o_ref):
  o_ref[...] = jnp.tanh(_dot_t(x_ref[...], w_ref[...]) + b_ref[...])


def _layer_body(agg_ref, h_ref, wrel_ref, wroot_ref, b_ref, o_ref):
  a = agg_ref[0] + agg_ref[1]
  o_ref[...] = jnp.tanh((_dot_t(a, wrel_ref[...]) + b_ref[...]) +
                        _dot_t(h_ref[...], wroot_ref[...]))


def _final_body(h_ref, w_ref, b_ref, o_ref):
  o_ref[...] = jnp.maximum(
      _dot_t(h_ref[...], w_ref[...]) + b_ref[...], 0.0)


def _row_blocked(body, n_pad, d, in_specs):
  grid = (n_pad // _ROW_BLK,)
  return pl.pallas_call(
      body,
      grid=grid,
      in_specs=in_specs,
      out_specs=pl.BlockSpec((_ROW_BLK, d), lambda i: (i, 0)),
      out_shape=jax.ShapeDtypeStruct((n_pad, d), jnp.float32),
  )


def _mat_spec(d):
  return pl.BlockSpec((d, d), lambda i: (0, 0))


def _bias_spec(d):
  return pl.BlockSpec((1, d), lambda i: (0, 0))


def kernel(x, edge_index, W1, b1, Wrel, brel, Wroot, W2, b2):
  n, d = x.shape
  e = edge_index.shape[1]
  l = Wrel.shape[0]

  n_pad = _cdiv(n, _NS * _CH) * _NS * _CH
  n_chunks = _cdiv(e, _NW * _CH)
  t_edges = n_chunks * _CH
  n_acc = _cdiv(n + 2 + 3 * _NW, _NS * 8) * _NS * 8
  if n_acc > n_pad:
    n_pad = _cdiv(n_acc, _NS * _CH) * _NS * _CH

  x_pad = jnp.pad(x, ((0, n_pad - n), (0, 0)))

  order = jnp.argsort(edge_index[1], stable=True)
  ds = edge_index[1][order]
  ss = edge_index[0][order]

  base = e // _NW
  slack = t_edges - base
  q = jnp.arange(_NW, dtype=jnp.int32) * base
  rs = jnp.searchsorted(ds, ds[q], side="left").astype(jnp.int32)
  starts = jnp.where(q - rs <= slack - 1, rs, q)
  ends = jnp.concatenate([starts[1:], jnp.array([e], jnp.int32)])
  idx2 = starts[:, None] + jnp.arange(t_edges, dtype=jnp.int32)[None, :]
  valid = idx2 < ends[:, None]
  idxc = jnp.minimum(idx2, e - 1)
  d2 = jnp.where(valid, ds[idxc], n)
  s2 = jnp.where(valid, ss[idxc], 0)
  col = jnp.arange(t_edges, dtype=jnp.int32)[None, :]
  prev = jnp.roll(d2, 1, axis=1)
  nxt = jnp.roll(d2, -1, axis=1)
  same2 = ((d2 == prev) & (col != 0)).astype(jnp.int32)
  is_end = (d2 != nxt) | (col == t_edges - 1)
  tile_id = jnp.arange(_NW, dtype=jnp.int32)[:, None]
  sink2 = n + 1 + tile_id * 3 + (col % 3)
  sdst2 = jnp.where(is_end, d2, sink2)

  meta3 = jnp.stack(
      [s2.reshape(_NW, n_chunks, _CH),
       sdst2.reshape(_NW, n_chunks, _CH),
       same2.reshape(_NW, n_chunks, _CH)],
      axis=2).reshape(_NW, n_chunks * 3, _CH)

  sc_agg = _make_sc_agg(n_pad, n_acc, n_chunks, d)

  row_spec = pl.BlockSpec((_ROW_BLK, d), lambda i: (i, 0))
  agg_spec = pl.BlockSpec((_NC, _ROW_BLK, d), lambda i: (0, i, 0))

  dense0 = _row_blocked(_dense0_body, n_pad, d,
                        [row_spec, _mat_spec(d), _bias_spec(d)])
  layer = _row_blocked(_layer_body, n_pad, d,
                       [agg_spec, row_spec, _mat_spec(d), _mat_spec(d),
                        _bias_spec(d)])
  final = _row_blocked(_final_body, n_pad, d,
                       [row_spec, _mat_spec(d), _bias_spec(d)])

  h = dense0(x_pad, W1, b1.reshape(1, d))
  for i in range(l):
    agg = sc_agg(h, meta3)
    h = layer(agg, h, Wrel[i], Wroot[i], brel[i].reshape(1, d))
  out = final(h, W2, b2.reshape(1, d))
  return out[:n]
